# baseline (speedup 1.0000x reference)
import jax
import jax.numpy as jnp
from jax import lax
from jax.experimental import pallas as pl
from jax.experimental.pallas import tpu as pltpu
from jax.experimental.pallas import tpu_sc as plsc

BATCH = 4096
SEQ = 200
DIM = 64
N = BATCH * SEQ

NUM_CORES = 2
NUM_SUBCORES = 16
NW = NUM_CORES * NUM_SUBCORES
ROWS_PER_W = N // NW

CHUNK = 640
G = ROWS_PER_W // CHUNK
NBUF = 3

TMPL = (SEQ + CHUNK + 7) // 8 * 8


def _body(xf_hbm, table_hbm, tmpl_hbm, out_hbm,
          iv0, iv1, iv2, b0, b1, b2,
          si0, si1, si2, sg0, sg1, sg2, ss0, ss1, ss2):
    idx_v = [iv0, iv1, iv2]
    buf = [b0, b1, b2]
    sem_in = [si0, si1, si2]
    sem_g = [sg0, sg1, sg2]
    sem_s = [ss0, ss1, ss2]

    wid = lax.axis_index("s") * NUM_CORES + lax.axis_index("c")
    base0 = wid * ROWS_PER_W

    def in_copies(g, b):
        base = pl.multiple_of(base0 + g * CHUNK, CHUNK)
        tmpl_off = pl.multiple_of(lax.rem(g * CHUNK, SEQ), 8)
        return (
            pltpu.make_async_copy(xf_hbm.at[pl.ds(base, CHUNK)], idx_v[b], sem_in[b]),
            pltpu.make_async_copy(tmpl_hbm.at[pl.ds(tmpl_off, CHUNK)], buf[b], sem_in[b]),
        )

    def gather_copy(b):
        return pltpu.make_async_copy(table_hbm.at[idx_v[b]], buf[b], sem_g[b])

    def out_copy(g, b):
        base = pl.multiple_of(base0 + g * CHUNK, CHUNK)
        return pltpu.make_async_copy(buf[b], out_hbm.at[pl.ds(base, CHUNK)], sem_s[b])

    def step(g, b, bn, recycle, prefetch):
        for c in in_copies(g, b):
            c.wait()
        gather_copy(b).start(add=True)
        if recycle:
            out_copy(g - (NBUF - 1), bn).wait()
        if prefetch:
            for c in in_copies(g + 1, bn):
                c.start()
        gather_copy(b).wait()
        out_copy(g, b).start()

    for c in in_copies(0, 0):
        c.start()
    step(0, 0, 1, False, True)
    step(1, 1, 2, False, True)

    RB = (G - 4) // NBUF
    def ring(blk, carry):
        g0 = 2 + blk * NBUF
        step(g0 + 0, 2, 0, True, True)
        step(g0 + 1, 0, 1, True, True)
        step(g0 + 2, 1, 2, True, True)
        return carry

    lax.fori_loop(0, RB, ring, 0)

    for g in range(2 + NBUF * RB, G):
        step(g, g % NBUF, (g + 1) % NBUF, True, g + 1 <= G - 1)
    for g in range(G - NBUF + 1, G):
        out_copy(g, g % NBUF).wait()


@jax.jit
def _run(xf, table, tmpl):
    mesh = plsc.VectorSubcoreMesh(core_axis_name="c", subcore_axis_name="s")
    f = pl.kernel(
        _body,
        out_type=jax.ShapeDtypeStruct((N, DIM), jnp.float32),
        mesh=mesh,
        compiler_params=pltpu.CompilerParams(use_tc_tiling_on_sc=False),
        scratch_types=(
            [pltpu.VMEM((CHUNK,), jnp.int32) for _ in range(NBUF)]
            + [pltpu.VMEM((CHUNK, DIM), jnp.float32) for _ in range(NBUF)]
            + [pltpu.SemaphoreType.DMA for _ in range(3 * NBUF)]
        ),
    )
    return f(xf, table, tmpl)


def kernel(x, table, pos_encoding):
    xf = x.reshape(N).astype(jnp.int32)
    reps = -(-TMPL // SEQ)
    tmpl = jnp.tile(pos_encoding[:SEQ], (reps, 1))[:TMPL]
    out = _run(xf, table, tmpl)
    return out.reshape(BATCH, SEQ, DIM)


# speedup vs baseline: 1.2658x; 1.2658x over previous
import jax
import jax.numpy as jnp
from jax import lax
from jax.experimental import pallas as pl
from jax.experimental.pallas import tpu as pltpu
from jax.experimental.pallas import tpu_sc as plsc

BATCH = 4096
SEQ = 200
DIM = 64
N = BATCH * SEQ

NUM_CORES = 2
NUM_SUBCORES = 16
NW = NUM_CORES * NUM_SUBCORES
ROWS_PER_W = N // NW

CHUNK = 800
G = ROWS_PER_W // CHUNK
REPS = CHUNK // SEQ

LANES = 16
JD = DIM // LANES


def _body(xf_hbm, table_hbm, pos_hbm, out_hbm,
          iv0, iv1, b0, b1, pos_t, sin0, sin1, sg0, sg1, ss0, ss1):
    idx_v = [iv0, iv1]
    buf = [b0, b1]
    sem_in = [sin0, sin1]
    sem_g = [sg0, sg1]
    sem_s = [ss0, ss1]

    wid = lax.axis_index("s") * NUM_CORES + lax.axis_index("c")
    base0 = wid * ROWS_PER_W

    pltpu.sync_copy(pos_hbm.at[pl.ds(0, SEQ)], pos_t)

    def idx_copy(g, b):
        base = pl.multiple_of(base0 + g * CHUNK, CHUNK)
        return pltpu.make_async_copy(xf_hbm.at[pl.ds(base, CHUNK)], idx_v[b], sem_in[b])

    def gather_copy(b):
        return pltpu.make_async_copy(table_hbm.at[idx_v[b]], buf[b], sem_g[b])

    def stage_in(g, b):
        idx_copy(g, b).start()
        idx_copy(g, b).wait()
        gather_copy(b).start()

    def out_copy(g, b):
        base = pl.multiple_of(base0 + g * CHUNK, CHUNK)
        return pltpu.make_async_copy(buf[b], out_hbm.at[pl.ds(base, CHUNK)], sem_s[b])

    def add_pos(b):
        def srow(s, carry):
            for j in range(JD):
                v = pos_t[s, pl.ds(j * LANES, LANES)]
                for q in range(REPS):
                    plsc.addupdate(buf[b].at[q * SEQ + s, pl.ds(j * LANES, LANES)], v)
            return carry

        lax.fori_loop(0, SEQ, srow, 0)

    def step(g, b, o, first, last):
        gather_copy(b).wait()
        if not last:
            if not first:
                out_copy(g - 1, o).wait()
            stage_in(g + 1, o)
        add_pos(b)
        out_copy(g, b).start()

    stage_in(0, 0)
    step(0, 0, 1, True, False)

    def pair(blk, carry):
        g0 = 1 + blk * 2
        step(g0, 1, 0, False, False)
        step(g0 + 1, 0, 1, False, False)
        return carry

    lax.fori_loop(0, (G - 2) // 2, pair, 0)

    step(G - 1, 1, 0, False, True)
    out_copy(G - 2, 0).wait()
    out_copy(G - 1, 1).wait()


@jax.jit
def _run(xf, table, pos):
    mesh = plsc.VectorSubcoreMesh(core_axis_name="c", subcore_axis_name="s")
    f = pl.kernel(
        _body,
        out_type=jax.ShapeDtypeStruct((N, DIM), jnp.float32),
        mesh=mesh,
        compiler_params=pltpu.CompilerParams(use_tc_tiling_on_sc=False),
        scratch_types=(
            [pltpu.VMEM((CHUNK,), jnp.int32) for _ in range(2)]
            + [pltpu.VMEM((CHUNK, DIM), jnp.float32) for _ in range(2)]
            + [pltpu.VMEM((SEQ, DIM), jnp.float32)]
            + [pltpu.SemaphoreType.DMA for _ in range(6)]
        ),
    )
    return f(xf, table, pos)


def kernel(x, table, pos_encoding):
    xf = x.reshape(N).astype(jnp.int32)
    out = _run(xf, table, pos_encoding)
    return out.reshape(BATCH, SEQ, DIM)
